# baseline (speedup 1.0000x reference)
import functools

import jax
import jax.numpy as jnp
from jax import lax
from jax.experimental import pallas as pl
from jax.experimental.pallas import tpu as pltpu
from jax.experimental.pallas import tpu_sc as plsc

BATCH = 16384
EMB_DIM = 32
NUM_CORES = 2
NUM_SUBCORES = 16
LANES = 16
NUM_WORKERS = NUM_CORES * NUM_SUBCORES
BPW = BATCH // NUM_WORKERS
CHUNK = 128
NCHUNK = BPW // CHUNK


def _mf_body(u_hbm, v_hbm, ut_hbm, vt_hbm, out_hbm,
             uidx_v, vidx_v, ue_v, ve_v, out_v, usem, vsem):
    wid = lax.axis_index("s") * NUM_CORES + lax.axis_index("c")
    base = wid * BPW

    pltpu.sync_copy(u_hbm.at[pl.ds(base, BPW)], uidx_v)
    pltpu.sync_copy(v_hbm.at[pl.ds(base, BPW)], vidx_v)

    copies = []
    for i in range(NCHUNK):
        sl = pl.ds(i * CHUNK, CHUNK)
        copies.append(pltpu.async_copy(ut_hbm.at[uidx_v.at[sl]], ue_v.at[sl], usem))
        copies.append(pltpu.async_copy(vt_hbm.at[vidx_v.at[sl]], ve_v.at[sl], vsem))
    for c in copies:
        c.wait()

    row_iota = lax.iota(jnp.int32, LANES)

    def block(j, carry):
        rows = j * LANES + row_iota
        acc = jnp.zeros((LANES,), jnp.float32)
        for d in range(EMB_DIM):
            col = jnp.full((LANES,), d, jnp.int32)
            ue = plsc.load_gather(ue_v, [rows, col])
            ve = plsc.load_gather(ve_v, [rows, col])
            acc = acc + ue * ve
        out_v[pl.ds(j * LANES, LANES)] = 4.0 / (1.0 + jnp.exp(-acc)) + 1.0
        return carry

    lax.fori_loop(0, BPW // LANES, block, 0)

    pltpu.sync_copy(out_v, out_hbm.at[pl.ds(base, BPW)])


def kernel(u, v, u_table, v_table):
    mesh = plsc.VectorSubcoreMesh(core_axis_name="c", subcore_axis_name="s")
    k = pl.kernel(
        _mf_body,
        mesh=mesh,
        compiler_params=pltpu.CompilerParams(
            needs_layout_passes=False, use_tc_tiling_on_sc=False),
        out_type=jax.ShapeDtypeStruct((BATCH,), jnp.float32),
        scratch_types=[
            pltpu.VMEM((BPW,), jnp.int32),
            pltpu.VMEM((BPW,), jnp.int32),
            pltpu.VMEM((BPW, EMB_DIM), jnp.float32),
            pltpu.VMEM((BPW, EMB_DIM), jnp.float32),
            pltpu.VMEM((BPW,), jnp.float32),
            pltpu.SemaphoreType.DMA,
            pltpu.SemaphoreType.DMA,
        ],
    )
    return k(u, v, u_table, v_table)


# speedup vs baseline: 3.1540x; 3.1540x over previous
import jax
import jax.numpy as jnp
from jax import lax
from jax.experimental import pallas as pl
from jax.experimental.pallas import tpu as pltpu
from jax.experimental.pallas import tpu_sc as plsc

BATCH = 16384
EMB_DIM = 32
NUM_CORES = 2
NUM_SUBCORES = 16
LANES = 16
NUM_WORKERS = NUM_CORES * NUM_SUBCORES
BPW = BATCH // NUM_WORKERS
WAVE = 8
PANEL = 128


def _mf_body(u_hbm, v_hbm, ut_hbm, vt_hbm, out_hbm,
             uidx_v, vidx_v, upan_v, vpan_v, ucol_v, vcol_v, out_v,
             usem, vsem):
    wid = lax.axis_index("s") * NUM_CORES + lax.axis_index("c")
    base = wid * BPW

    pltpu.sync_copy(u_hbm.at[pl.ds(base, BPW)], uidx_v)
    pltpu.sync_copy(v_hbm.at[pl.ds(base, BPW)], vidx_v)

    dlo = lax.iota(jnp.int32, LANES)
    dhi = dlo + LANES

    def wave16(j, carry):
        uvec = uidx_v[pl.ds(j * 16, 16)]
        vvec = vidx_v[pl.ds(j * 16, 16)]
        for half in range(2):
            copies = []
            cs_u = []
            cs_v = []
            for k in range(WAVE):
                lane = half * WAVE + k
                ru = uvec[lane]
                rv = vvec[lane]
                cu = lax.rem(ru, PANEL)
                cv = lax.rem(rv, PANEL)
                pu = pl.multiple_of(ru - cu, PANEL)
                pv = pl.multiple_of(rv - cv, PANEL)
                cs_u.append(cu)
                cs_v.append(cv)
                copies.append(pltpu.async_copy(
                    ut_hbm.at[:, pl.ds(pu, PANEL)], upan_v.at[k], usem))
                copies.append(pltpu.async_copy(
                    vt_hbm.at[:, pl.ds(pv, PANEL)], vpan_v.at[k], vsem))
            for c in copies:
                c.wait()
            for k in range(WAVE):
                lane = half * WAVE + k
                col = j * 16 + lane
                kf = jnp.full((LANES,), k, jnp.int32)
                cuf = jnp.full((LANES,), cs_u[k], jnp.int32)
                cvf = jnp.full((LANES,), cs_v[k], jnp.int32)
                colf = jnp.full((LANES,), col, jnp.int32)
                for dvec in (dlo, dhi):
                    uvals = plsc.load_gather(upan_v, [kf, dvec, cuf])
                    plsc.store_scatter(ucol_v, [dvec, colf], uvals)
                    vvals = plsc.load_gather(vpan_v, [kf, dvec, cvf])
                    plsc.store_scatter(vcol_v, [dvec, colf], vvals)
        return carry

    lax.fori_loop(0, BPW // 16, wave16, 0)

    def block(c, carry):
        col = c * LANES
        acc = jnp.zeros((LANES,), jnp.float32)
        for d in range(EMB_DIM):
            acc = acc + ucol_v[d, pl.ds(col, LANES)] * vcol_v[d, pl.ds(col, LANES)]
        out_v[pl.ds(col, LANES)] = 4.0 / (1.0 + jnp.exp(-acc)) + 1.0
        return carry

    lax.fori_loop(0, BPW // LANES, block, 0)

    pltpu.sync_copy(out_v, out_hbm.at[pl.ds(base, BPW)])


def kernel(u, v, u_table, v_table):
    mesh = plsc.VectorSubcoreMesh(core_axis_name="c", subcore_axis_name="s")
    k = pl.kernel(
        _mf_body,
        mesh=mesh,
        compiler_params=pltpu.CompilerParams(
            needs_layout_passes=False, use_tc_tiling_on_sc=True),
        out_type=jax.ShapeDtypeStruct((BATCH,), jnp.float32),
        scratch_types=[
            pltpu.VMEM((BPW,), jnp.int32),
            pltpu.VMEM((BPW,), jnp.int32),
            pltpu.VMEM((WAVE, EMB_DIM, PANEL), jnp.float32),
            pltpu.VMEM((WAVE, EMB_DIM, PANEL), jnp.float32),
            pltpu.VMEM((EMB_DIM, BPW), jnp.float32),
            pltpu.VMEM((EMB_DIM, BPW), jnp.float32),
            pltpu.VMEM((BPW,), jnp.float32),
            pltpu.SemaphoreType.DMA,
            pltpu.SemaphoreType.DMA,
        ],
    )
    return k(u, v, u_table.T, v_table.T)


# speedup vs baseline: 3.7745x; 1.1968x over previous
import jax
import jax.numpy as jnp
from jax import lax
from jax.experimental import pallas as pl
from jax.experimental.pallas import tpu as pltpu
from jax.experimental.pallas import tpu_sc as plsc

BATCH = 16384
EMB_DIM = 32
NUM_CORES = 2
NUM_SUBCORES = 16
LANES = 16
NUM_WORKERS = NUM_CORES * NUM_SUBCORES
BPW = BATCH // NUM_WORKERS
WAVE = 4
NWAVES = BPW // WAVE
PANEL = 128
IPAD = BPW + LANES


def _mf_body(u_hbm, v_hbm, ut_hbm, vt_hbm, out_hbm,
             uidx_v, vidx_v, upanb_v, vpanb_v, ucolb_v, vcolb_v,
             upan_v, vpan_v, ucol_v, vcol_v, out_v,
             usem0, vsem0, usem1, vsem1):
    wid = lax.axis_index("s") * NUM_CORES + lax.axis_index("c")
    base = wid * BPW

    pltpu.sync_copy(u_hbm.at[pl.ds(base, BPW)], uidx_v)
    pltpu.sync_copy(v_hbm.at[pl.ds(base, BPW)], vidx_v)

    zeros16 = jnp.zeros((LANES,), jnp.int32)

    def prep(t, carry):
        sl = pl.ds(t * LANES, LANES)
        uvec = uidx_v[sl]
        vvec = vidx_v[sl]
        upanb_v[sl] = uvec - lax.rem(uvec, PANEL)
        ucolb_v[sl] = lax.rem(uvec, PANEL)
        vpanb_v[sl] = vvec - lax.rem(vvec, PANEL)
        vcolb_v[sl] = lax.rem(vvec, PANEL)
        return carry

    lax.fori_loop(0, BPW // LANES, prep, 0)
    upanb_v[pl.ds(BPW, LANES)] = zeros16
    vpanb_v[pl.ds(BPW, LANES)] = zeros16
    ucolb_v[pl.ds(BPW, LANES)] = zeros16
    vcolb_v[pl.ds(BPW, LANES)] = zeros16

    def fire(j, parity):
        upans = upanb_v[pl.ds(j * WAVE, LANES)]
        vpans = vpanb_v[pl.ds(j * WAVE, LANES)]
        usem = usem0 if parity == 0 else usem1
        vsem = vsem0 if parity == 0 else vsem1
        for k in range(WAVE):
            slot = parity * WAVE + k
            pu = pl.multiple_of(upans[k], PANEL)
            pv = pl.multiple_of(vpans[k], PANEL)
            pltpu.async_copy(ut_hbm.at[:, pl.ds(pu, PANEL)],
                             upan_v.at[slot], usem)
            pltpu.async_copy(vt_hbm.at[:, pl.ds(pv, PANEL)],
                             vpan_v.at[slot], vsem)

    dlo = lax.iota(jnp.int32, LANES)
    dhi = dlo + LANES

    def drain_and_extract(j, parity):
        usem = usem0 if parity == 0 else usem1
        vsem = vsem0 if parity == 0 else vsem1
        for _ in range(WAVE):
            pltpu.make_async_copy(ut_hbm.at[:, pl.ds(0, PANEL)],
                                  upan_v.at[0], usem).wait()
            pltpu.make_async_copy(vt_hbm.at[:, pl.ds(0, PANEL)],
                                  vpan_v.at[0], vsem).wait()
        ucols = ucolb_v[pl.ds(j * WAVE, LANES)]
        vcols = vcolb_v[pl.ds(j * WAVE, LANES)]
        for k in range(WAVE):
            slot = parity * WAVE + k
            col = j * WAVE + k
            sf = jnp.full((LANES,), slot, jnp.int32)
            cuf = jnp.full((LANES,), ucols[k], jnp.int32)
            cvf = jnp.full((LANES,), vcols[k], jnp.int32)
            colf = jnp.full((LANES,), col, jnp.int32)
            for dvec in (dlo, dhi):
                uvals = plsc.load_gather(upan_v, [sf, dvec, cuf])
                plsc.store_scatter(ucol_v, [dvec, colf], uvals)
                vvals = plsc.load_gather(vpan_v, [sf, dvec, cvf])
                plsc.store_scatter(vcol_v, [dvec, colf], vvals)

    fire(0, 0)

    def pair(p, carry):
        j0 = p * 2
        fire(j0 + 1, 1)
        drain_and_extract(j0, 0)

        @pl.when(p < NWAVES // 2 - 1)
        def _():
            fire(j0 + 2, 0)

        drain_and_extract(j0 + 1, 1)
        return carry

    lax.fori_loop(0, NWAVES // 2, pair, 0)

    def block(c, carry):
        col = c * LANES
        acc = jnp.zeros((LANES,), jnp.float32)
        for d in range(EMB_DIM):
            acc = acc + ucol_v[d, pl.ds(col, LANES)] * vcol_v[d, pl.ds(col, LANES)]
        out_v[pl.ds(col, LANES)] = 4.0 / (1.0 + jnp.exp(-acc)) + 1.0
        return carry

    lax.fori_loop(0, BPW // LANES, block, 0)

    pltpu.sync_copy(out_v, out_hbm.at[pl.ds(base, BPW)])


def kernel(u, v, u_table, v_table):
    mesh = plsc.VectorSubcoreMesh(core_axis_name="c", subcore_axis_name="s")
    k = pl.kernel(
        _mf_body,
        mesh=mesh,
        compiler_params=pltpu.CompilerParams(
            needs_layout_passes=False, use_tc_tiling_on_sc=True),
        out_type=jax.ShapeDtypeStruct((BATCH,), jnp.float32),
        scratch_types=[
            pltpu.VMEM((BPW,), jnp.int32),
            pltpu.VMEM((BPW,), jnp.int32),
            pltpu.VMEM((IPAD,), jnp.int32),
            pltpu.VMEM((IPAD,), jnp.int32),
            pltpu.VMEM((IPAD,), jnp.int32),
            pltpu.VMEM((IPAD,), jnp.int32),
            pltpu.VMEM((2 * WAVE, EMB_DIM, PANEL), jnp.float32),
            pltpu.VMEM((2 * WAVE, EMB_DIM, PANEL), jnp.float32),
            pltpu.VMEM((EMB_DIM, BPW), jnp.float32),
            pltpu.VMEM((EMB_DIM, BPW), jnp.float32),
            pltpu.VMEM((BPW,), jnp.float32),
            pltpu.SemaphoreType.DMA,
            pltpu.SemaphoreType.DMA,
            pltpu.SemaphoreType.DMA,
            pltpu.SemaphoreType.DMA,
        ],
    )
    return k(u, v, u_table.T, v_table.T)
